# baseline (speedup 1.0000x reference)
import functools

import jax
import jax.numpy as jnp
from jax import lax
from jax.experimental import pallas as pl
from jax.experimental.pallas import tpu as pltpu
from jax.experimental.pallas import tpu_sc as plsc

_info = plsc.get_sparse_core_info()
_NC, _NS = _info.num_cores, _info.num_subcores
_NW = _NC * _NS


def _iota16():
    return lax.iota(jnp.int32, 16)


def _make_repack(V, D):
    assert D == 64 and V % 128 == 64
    n_chunks = V // 128
    n_full = (n_chunks // _NW) * _NW
    mesh = plsc.VectorSubcoreMesh(core_axis_name="c", subcore_axis_name="s")

    @functools.partial(
        pl.kernel,
        mesh=mesh,
        out_type=jax.ShapeDtypeStruct((V // 2, 128), jnp.float32),
        scratch_types=[
            pltpu.VMEM((2, D, 128), jnp.float32),
            pltpu.VMEM((2, D, 128), jnp.float32),
            pltpu.VMEM((D, 64), jnp.float32),
            pltpu.SemaphoreType.DMA((2,)),
            pltpu.SemaphoreType.DMA((2,)),
        ],
        compiler_params=pltpu.CompilerParams(needs_layout_passes=False),
    )
    def repack(tt_hbm, t2_hbm, a_v, b_v, c_v, rsem, wsem):
        wid = lax.axis_index("s") * _NC + lax.axis_index("c")
        n_iter = n_full // _NW

        def rd(i, p):
            k = i * _NW + wid
            return pltpu.make_async_copy(
                tt_hbm.at[:, pl.ds(k * 128, 128)], a_v.at[p], rsem.at[p]
            )

        def wr(i, p):
            k = i * _NW + wid
            return pltpu.make_async_copy(
                b_v.at[p], t2_hbm.at[pl.ds(k * 64, 64)], wsem.at[p]
            )

        def transpose(p):
            rows = [_iota16() + 16 * m for m in range(4)]

            def tbody(r, carry):
                for j in range(8):
                    col = jnp.full((16,), 2 * r + j // 4, jnp.int32)
                    vals = plsc.load_gather(a_v.at[p], [rows[j % 4], col])
                    b_v[p, r, pl.ds(16 * j, 16)] = vals
                return carry

            lax.fori_loop(0, D, tbody, 0)

        rd(0, 0).start()

        def body(i, carry):
            p = lax.rem(i, 2)
            q = 1 - p
            rd(i, p).wait()

            @pl.when(i + 1 < n_iter)
            def _():
                rd(i + 1, q).start()

            @pl.when(i >= 2)
            def _():
                wr(i - 2, p).wait()

            transpose(p)
            wr(i, p).start()
            return carry

        lax.fori_loop(0, n_iter, body, 0)

        @pl.when(n_iter >= 2)
        def _():
            wr(n_iter - 2, lax.rem(n_iter, 2)).wait()

        wr(n_iter - 1, lax.rem(n_iter - 1, 2)).wait()

        n_tail = n_chunks - n_full

        @pl.when(wid < n_tail)
        def _():
            k = n_full + wid
            pltpu.make_async_copy(
                tt_hbm.at[:, pl.ds(k * 128, 128)], a_v.at[0], rsem.at[0]
            ).start()
            pltpu.make_async_copy(
                tt_hbm.at[:, pl.ds(k * 128, 128)], a_v.at[0], rsem.at[0]
            ).wait()
            transpose(0)
            pltpu.make_async_copy(
                b_v.at[0], t2_hbm.at[pl.ds(k * 64, 64)], wsem.at[0]
            ).start()
            pltpu.make_async_copy(
                b_v.at[0], t2_hbm.at[pl.ds(k * 64, 64)], wsem.at[0]
            ).wait()

        @pl.when(wid == n_tail)
        def _():
            v0 = n_chunks * 128
            pltpu.make_async_copy(
                tt_hbm.at[:, pl.ds(v0, 64)], c_v, rsem.at[0]
            ).start()
            pltpu.make_async_copy(
                tt_hbm.at[:, pl.ds(v0, 64)], c_v, rsem.at[0]
            ).wait()
            rows = [_iota16() + 16 * m for m in range(4)]

            def tbody(r, carry):
                for j in range(8):
                    col = jnp.full((16,), 2 * r + j // 4, jnp.int32)
                    vals = plsc.load_gather(c_v, [rows[j % 4], col])
                    b_v[0, r, pl.ds(16 * j, 16)] = vals
                return carry

            lax.fori_loop(0, 32, tbody, 0)
            pltpu.make_async_copy(
                b_v.at[0, pl.ds(0, 32)],
                t2_hbm.at[pl.ds(v0 // 2, 32)],
                wsem.at[0],
            ).start()
            pltpu.make_async_copy(
                b_v.at[0, pl.ds(0, 32)],
                t2_hbm.at[pl.ds(v0 // 2, 32)],
                wsem.at[0],
            ).wait()

    return repack


def _make_gather(B0, B1, V):
    assert B0 % (128 * _NW) == 0 and B1 % 8 == 0
    n_blocks = B1 // 8
    D = 64
    mesh = plsc.VectorSubcoreMesh(core_axis_name="c", subcore_axis_name="s")

    @functools.partial(
        pl.kernel,
        mesh=mesh,
        out_type=jax.ShapeDtypeStruct((B1, D, B0), jnp.float32),
        scratch_types=[
            pltpu.VMEM((2, 8, 128), jnp.int32),
            pltpu.VMEM((2, 8, 128), jnp.int32),
            pltpu.VMEM((2, 8, 8, 16), jnp.int32),
            pltpu.VMEM((2, 128, 128), jnp.float32),
            pltpu.VMEM((2, D, 128), jnp.float32),
            pltpu.SemaphoreType.DMA((2,)),
            pltpu.SemaphoreType.DMA((2,)),
            pltpu.SemaphoreType.DMA((2,)),
        ],
        compiler_params=pltpu.CompilerParams(needs_layout_passes=False),
    )
    def gather(xt_hbm, t2_hbm, out_hbm, xblk_v, idx2_v, parb_v, g_v, b_v,
               isem, gsem, ssem):
        wid = lax.axis_index("s") * _NC + lax.axis_index("c")
        lane0 = wid * (B0 // _NW)

        def ld_idx(ib, pb):
            return pltpu.make_async_copy(
                xt_hbm.at[pl.ds(ib * 8, 8), pl.ds(lane0, 128)],
                xblk_v.at[pb],
                isem.at[pb],
            )

        def prep(pb):
            def pbody(r, carry):
                for t in range(8):
                    v = xblk_v[pb, r, pl.ds(16 * t, 16)]
                    idx2_v[pb, r, pl.ds(16 * t, 16)] = v >> 1
                    parb_v[pb, r, t] = (v & 1) * 64
                return carry

            lax.fori_loop(0, 8, pbody, 0)

        def g_copy(pb, r, pg):
            return pltpu.make_async_copy(
                t2_hbm.at[idx2_v.at[pb, r]], g_v.at[pg], gsem.at[pg]
            )

        def s_copy(j, pg):
            return pltpu.make_async_copy(
                b_v.at[pg], out_hbm.at[j, :, pl.ds(lane0, 128)], ssem.at[pg]
            )

        def transpose(pb, r, pg):
            rows = [_iota16() + 16 * t for t in range(8)]
            pars = [parb_v[pb, r, t] for t in range(8)]

            def tbody(d, carry):
                for t in range(8):
                    vals = plsc.load_gather(g_v.at[pg], [rows[t], pars[t] + d])
                    b_v[pg, d, pl.ds(16 * t, 16)] = vals
                return carry

            lax.fori_loop(0, D, tbody, 0)

        ld_idx(0, 0).start()
        ld_idx(0, 0).wait()
        prep(0)
        ld_idx(1, 1).start()
        g_copy(0, 0, 0).start()

        def block(ib, carry):
            pb = lax.rem(ib, 2)
            qb = 1 - pb

            @pl.when(ib + 1 < n_blocks)
            def _():
                ld_idx(ib + 1, qb).wait()
                prep(qb)

            @pl.when(ib + 2 < n_blocks)
            def _():
                ld_idx(ib + 2, pb).start()

            def item(r, carry2):
                j = ib * 8 + r
                pg = lax.rem(j, 2)
                qg = 1 - pg
                g_copy(pb, r, pg).wait()

                @pl.when(r < 7)
                def _():
                    g_copy(pb, r + 1, qg).start()

                @pl.when((r == 7) & (ib + 1 < n_blocks))
                def _():
                    g_copy(qb, 0, qg).start()

                @pl.when(j >= 2)
                def _():
                    s_copy(j - 2, pg).wait()

                transpose(pb, r, pg)
                s_copy(j, pg).start()
                return carry2

            lax.fori_loop(0, 8, item, 0)
            return carry

        lax.fori_loop(0, n_blocks, block, 0)

        s_copy(B1 - 2, 0).wait()
        s_copy(B1 - 1, 1).wait()

    return gather


def kernel(x, table):
    B0, B1 = x.shape
    V, D = table.shape
    tt = table.T
    xt = x.T.astype(jnp.int32)
    t2 = _make_repack(V, D)(tt)
    out_t = _make_gather(B0, B1, V)(xt, t2)
    return out_t.transpose(2, 0, 1)


# speedup vs baseline: 2.4450x; 2.4450x over previous
import functools

import jax
import jax.numpy as jnp
from jax import lax
from jax.experimental import pallas as pl
from jax.experimental.pallas import tpu as pltpu
from jax.experimental.pallas import tpu_sc as plsc

_info = plsc.get_sparse_core_info()
_NC, _NS = _info.num_cores, _info.num_subcores
_NW = _NC * _NS


def _make_lookup(B0, B1, DP):
    assert B0 % _NW == 0
    n_iter = B0 // _NW
    b_per_w = n_iter * B1
    D = 64
    mesh = plsc.VectorSubcoreMesh(core_axis_name="c", subcore_axis_name="s")

    @functools.partial(
        pl.kernel,
        mesh=mesh,
        out_type=jax.ShapeDtypeStruct((B0, B1, D), jnp.float32),
        scratch_types=[
            pltpu.VMEM((b_per_w,), jnp.int32),
            pltpu.VMEM((2, B1, DP), jnp.float32),
            pltpu.VMEM((2, B1, D), jnp.float32),
            pltpu.SemaphoreType.DMA((2,)),
            pltpu.SemaphoreType.DMA((2,)),
        ],
    )
    def lookup(x_hbm, table_hbm, out_hbm, idx_v, rows_v, rows64_v, gsem, ssem):
        wid = lax.axis_index("s") * _NC + lax.axis_index("c")
        base = wid * n_iter

        pltpu.sync_copy(x_hbm.at[pl.ds(base * B1, b_per_w)], idx_v)

        def gather_parts(it, p):
            off = it * B1
            yield idx_v.at[pl.ds(off, 128)], rows_v.at[p, pl.ds(0, 128)]
            yield (
                idx_v.at[pl.ds(off + 128, B1 - 128)],
                rows_v.at[p, pl.ds(128, B1 - 128)],
            )

        def start_gathers(it, p):
            for isl, rsl in gather_parts(it, p):
                pltpu.make_async_copy(table_hbm.at[isl], rsl, gsem.at[p]).start()

        def wait_gathers(it, p):
            for isl, rsl in gather_parts(it, p):
                pltpu.make_async_copy(table_hbm.at[isl], rsl, gsem.at[p]).wait()

        def compact(p):
            def cbody(j, carry):
                for r in range(8):
                    for k in range(D // 16):
                        rows64_v[p, j * 8 + r, pl.ds(k * 16, 16)] = rows_v[
                            p, j * 8 + r, pl.ds(k * 16, 16)
                        ]
                return carry

            lax.fori_loop(0, B1 // 8, cbody, 0)

        def start_store(it, p):
            pltpu.make_async_copy(
                rows64_v.at[p], out_hbm.at[base + it], ssem.at[p]
            ).start()

        def wait_store(it, p):
            pltpu.make_async_copy(
                rows64_v.at[p], out_hbm.at[base + it], ssem.at[p]
            ).wait()

        def step(it, p):
            q = 1 - p
            wait_store(it - 1, q)
            start_gathers(it + 1, q)
            wait_gathers(it, p)
            compact(p)
            start_store(it, p)

        start_gathers(0, 0)
        start_gathers(1, 1)
        wait_gathers(0, 0)
        compact(0)
        start_store(0, 0)

        def body(i, carry):
            for p in (1, 0):
                step(2 * i + (1 if p == 1 else 2), p)
            return carry

        lax.fori_loop(0, (n_iter - 2) // 2, body, 0)

        it = n_iter - 1
        wait_store(it - 1, 0)
        wait_gathers(it, 1)
        compact(1)
        start_store(it, 1)
        wait_store(it, 1)

    return lookup


def kernel(x, table):
    B0, B1 = x.shape
    V, D = table.shape
    DP = 128
    tpad = jnp.pad(table, ((0, 0), (0, DP - D)))
    xflat = x.reshape((B0 * B1,)).astype(jnp.int32)
    return _make_lookup(B0, B1, DP)(xflat, tpad)
